# baseline (speedup 1.0000x reference)
import functools

import jax
import jax.numpy as jnp
from jax import lax
from jax.experimental import pallas as pl
from jax.experimental.pallas import tpu as pltpu
from jax.experimental.pallas import tpu_sc as plsc

_EMB = 32
_BATCH = 16384
_HIST = 200
_B = _BATCH * _HIST
_NW = 32
_HT = _HIST // 2
_BH = _BATCH * _HT
_BPW = _BH // _NW
_C = 800
_G = _BPW // _C
_NBUF = 4
_K = 2
_PAD = 128

_mesh = plsc.VectorSubcoreMesh(core_axis_name="c", subcore_axis_name="s")


def _make_gather(t0):
    @functools.partial(
        pl.kernel,
        mesh=_mesh,
        out_type=jax.ShapeDtypeStruct((_BH, _PAD), jnp.float32),
        scratch_types=[
            pltpu.VMEM((_NBUF * _C,), jnp.int32),
            pltpu.VMEM((_NBUF, _C, _EMB), jnp.float32),
        ] + [pltpu.SemaphoreType.DMA] * (3 * _NBUF),
        compiler_params=pltpu.CompilerParams(use_tc_tiling_on_sc=False),
    )
    def _gather_padded(idx_hbm, table_hbm, out_hbm, idx_v, rows_v, *sems):
        sem_g = sems[:_NBUF]
        sem_w = sems[_NBUF:2 * _NBUF]
        sem_i = sems[2 * _NBUF:]
        wid = lax.axis_index("s") * 2 + lax.axis_index("c")
        src_base = t0 * _BATCH + wid * _BPW
        dst_base = wid * _BPW

        def prefetch_idx(j, b):
            pltpu.async_copy(
                idx_hbm.at[pl.ds(src_base + j * _C, _C)],
                idx_v.at[pl.ds(b * _C, _C)], sem_i[b],
            )

        def fire_gather(j, b):
            pltpu.make_async_copy(
                idx_hbm.at[pl.ds(src_base + j * _C, _C)],
                idx_v.at[pl.ds(b * _C, _C)], sem_i[b],
            ).wait()
            pltpu.async_copy(
                table_hbm.at[idx_v.at[pl.ds(b * _C, _C)]], rows_v.at[b], sem_g[b]
            )

        def wb_dst(i):
            return out_hbm.at[pl.ds(dst_base + i * _C, _C), pl.ds(0, _EMB)]

        for j in range(_K + 1):
            prefetch_idx(j, j)
        for j in range(_K):
            fire_gather(j, j)

        def group(gg, carry):
            for phase in range(_NBUF):
                i = gg * _NBUF + phase
                bi = phase
                bj = (phase + _K) % _NBUF
                bp = (phase + _K + 1) % _NBUF
                j = i + _K

                @pl.when(i + _K + 1 < _G)
                def _():
                    prefetch_idx(i + _K + 1, bp)

                @pl.when(j < _G)
                def _():
                    @pl.when(j >= _NBUF)
                    def _():
                        pltpu.make_async_copy(
                            rows_v.at[bj], wb_dst(0), sem_w[bj],
                        ).wait()

                    fire_gather(j, bj)

                pltpu.make_async_copy(
                    table_hbm.at[idx_v.at[pl.ds(bi * _C, _C)]], rows_v.at[bi],
                    sem_g[bi],
                ).wait()
                pltpu.async_copy(rows_v.at[bi], wb_dst(i), sem_w[bi])
            return carry

        lax.fori_loop(0, _G // _NBUF, group, 0)

        for b in range(_NBUF):
            pltpu.make_async_copy(rows_v.at[b], wb_dst(0), sem_w[b]).wait()

    return _gather_padded


_gather_lo = _make_gather(0)
_gather_hi = _make_gather(_HT)


def _transpose_body(x_ref, o_ref):
    xt = jnp.transpose(x_ref[0], (1, 0))
    o_ref[0] = xt[0:_EMB, :]


def _transpose_body2(x_ref, _prev_ref, o_ref):
    xt = jnp.transpose(x_ref[0], (1, 0))
    o_ref[0] = xt[0:_EMB, :]


_OUT3 = jax.ShapeDtypeStruct((_HIST, _EMB, _BATCH), jnp.float32)
_IN_SPEC = pl.BlockSpec((1, _BATCH, _PAD), lambda i: (i, 0, 0))


def _transpose_lo(y):
    return pl.pallas_call(
        _transpose_body,
        grid=(_HT,),
        in_specs=[_IN_SPEC],
        out_specs=pl.BlockSpec((1, _EMB, _BATCH), lambda i: (i, 0, 0)),
        out_shape=_OUT3,
    )(y)


def _transpose_hi(y, prev):
    return pl.pallas_call(
        _transpose_body2,
        grid=(_HT,),
        in_specs=[
            _IN_SPEC,
            pl.BlockSpec((1, _EMB, _BATCH), lambda i: (i + _HT, 0, 0)),
        ],
        out_specs=pl.BlockSpec((1, _EMB, _BATCH), lambda i: (i + _HT, 0, 0)),
        out_shape=_OUT3,
        input_output_aliases={1: 0},
    )(y, prev)


def kernel(input, weight):
    idx = jnp.transpose(input).reshape(_B).astype(jnp.int32)
    yp_lo = _gather_lo(idx, weight)
    yp_hi = _gather_hi(idx, weight)
    o_lo = _transpose_lo(yp_lo.reshape(_HT, _BATCH, _PAD))
    o = _transpose_hi(yp_hi.reshape(_HT, _BATCH, _PAD), o_lo)
    return o.transpose(2, 0, 1)


# speedup vs baseline: 1.4224x; 1.4224x over previous
import functools

import jax
import jax.numpy as jnp
from jax import lax
from jax.experimental import pallas as pl
from jax.experimental.pallas import tpu as pltpu
from jax.experimental.pallas import tpu_sc as plsc

_EMB = 32
_BATCH = 16384
_HIST = 200
_B = _BATCH * _HIST
_NW = 32
_BPW = _B // _NW
_C = 512
_G = _BPW // _C
_NBUF = 4
_K = 2
_TG = _HIST // 4

_mesh = plsc.VectorSubcoreMesh(core_axis_name="c", subcore_axis_name="s")


@functools.partial(
    pl.kernel,
    mesh=_mesh,
    out_type=jax.ShapeDtypeStruct((_TG, _BATCH, 4 * _EMB), jnp.float32),
    scratch_types=[
        pltpu.VMEM((_NBUF * _C,), jnp.int32),
        pltpu.VMEM((_NBUF, _C, _EMB), jnp.float32),
    ] + [pltpu.SemaphoreType.DMA] * (3 * _NBUF),
    compiler_params=pltpu.CompilerParams(use_tc_tiling_on_sc=False),
)
def _gather_interleaved(idx_hbm, table_hbm, out_hbm, idx_v, rows_v, *sems):
    sem_g = sems[:_NBUF]
    sem_w = sems[_NBUF:2 * _NBUF]
    sem_i = sems[2 * _NBUF:]
    wid = lax.axis_index("s") * 2 + lax.axis_index("c")
    base = wid * _BPW

    def prefetch_idx(j, b):
        pltpu.async_copy(
            idx_hbm.at[pl.ds(base + j * _C, _C)],
            idx_v.at[pl.ds(b * _C, _C)], sem_i[b],
        )

    def fire_gather(j, b):
        pltpu.make_async_copy(
            idx_hbm.at[pl.ds(base + j * _C, _C)],
            idx_v.at[pl.ds(b * _C, _C)], sem_i[b],
        ).wait()
        pltpu.async_copy(table_hbm.at[idx_v.at[pl.ds(b * _C, _C)]], rows_v.at[b], sem_g[b])

    def wb_dst(i):
        flat = base + i * _C
        t = flat // _BATCH
        b0 = flat % _BATCH
        return out_hbm.at[t // 4, pl.ds(b0, _C), pl.ds((t % 4) * _EMB, _EMB)]

    for j in range(_K + 1):
        prefetch_idx(j, j)
    for j in range(_K):
        fire_gather(j, j)

    def group(gg, carry):
        for phase in range(_NBUF):
            i = gg * _NBUF + phase
            bi = phase
            bj = (phase + _K) % _NBUF
            bp = (phase + _K + 1) % _NBUF
            j = i + _K

            @pl.when(i + _K + 1 < _G)
            def _():
                prefetch_idx(i + _K + 1, bp)

            @pl.when(j < _G)
            def _():
                @pl.when(j >= _NBUF)
                def _():
                    pltpu.make_async_copy(
                        rows_v.at[bj], wb_dst(0), sem_w[bj],
                    ).wait()

                fire_gather(j, bj)

            pltpu.make_async_copy(
                table_hbm.at[idx_v.at[pl.ds(bi * _C, _C)]], rows_v.at[bi], sem_g[bi]
            ).wait()
            pltpu.async_copy(rows_v.at[bi], wb_dst(i), sem_w[bi])
        return carry

    lax.fori_loop(0, _G // _NBUF, group, 0)

    for b in range(_NBUF):
        pltpu.make_async_copy(rows_v.at[b], wb_dst(0), sem_w[b]).wait()


def _transpose_body(x_ref, o_ref):
    xt = jnp.transpose(x_ref[0], (1, 0))
    o_ref[...] = xt.reshape(4, _EMB, _BATCH)


def _transpose_slabs(y):
    return pl.pallas_call(
        _transpose_body,
        grid=(_TG,),
        in_specs=[pl.BlockSpec((1, _BATCH, 4 * _EMB), lambda g: (g, 0, 0))],
        out_specs=pl.BlockSpec((4, _EMB, _BATCH), lambda g: (g, 0, 0)),
        out_shape=jax.ShapeDtypeStruct((_HIST, _EMB, _BATCH), jnp.float32),
    )(y)


def kernel(input, weight):
    idx = jnp.transpose(input).reshape(_B).astype(jnp.int32)
    z = _gather_interleaved(idx, weight)
    out3 = _transpose_slabs(z)
    return out3.transpose(2, 0, 1)


# speedup vs baseline: 1.4225x; 1.0001x over previous
import functools

import jax
import jax.numpy as jnp
from jax import lax
from jax.experimental import pallas as pl
from jax.experimental.pallas import tpu as pltpu
from jax.experimental.pallas import tpu_sc as plsc

_EMB = 32
_BATCH = 16384
_HIST = 200
_B = _BATCH * _HIST
_NW = 32
_BPW = _B // _NW
_C = 512
_G = _BPW // _C
_NBUF = 4
_K = 2
_TG = _HIST // 4

_mesh = plsc.VectorSubcoreMesh(core_axis_name="c", subcore_axis_name="s")


@functools.partial(
    pl.kernel,
    mesh=_mesh,
    out_type=jax.ShapeDtypeStruct((_TG * _BATCH, 4 * _EMB), jnp.float32),
    scratch_types=[
        pltpu.VMEM((_NBUF * _C,), jnp.int32),
        pltpu.VMEM((_NBUF, _C, _EMB), jnp.float32),
    ] + [pltpu.SemaphoreType.DMA] * (3 * _NBUF),
    compiler_params=pltpu.CompilerParams(use_tc_tiling_on_sc=False),
)
def _gather_interleaved(idx_hbm, table_hbm, out_hbm, idx_v, rows_v, *sems):
    sem_g = sems[:_NBUF]
    sem_w = sems[_NBUF:2 * _NBUF]
    sem_i = sems[2 * _NBUF:]
    wid = lax.axis_index("s") * 2 + lax.axis_index("c")
    base = wid * _BPW

    def prefetch_idx(j, b):
        pltpu.async_copy(
            idx_hbm.at[pl.ds(base + j * _C, _C)],
            idx_v.at[pl.ds(b * _C, _C)], sem_i[b],
        )

    def fire_gather(j, b):
        pltpu.make_async_copy(
            idx_hbm.at[pl.ds(base + j * _C, _C)],
            idx_v.at[pl.ds(b * _C, _C)], sem_i[b],
        ).wait()
        pltpu.async_copy(table_hbm.at[idx_v.at[pl.ds(b * _C, _C)]], rows_v.at[b], sem_g[b])

    def wb_start(i, b):
        flat = base + i * _C
        t = flat // _BATCH
        row0 = (t // 4) * _BATCH + flat % _BATCH
        tl = t % 4
        for w in range(4):
            @pl.when(tl == w)
            def _():
                pltpu.async_copy(
                    rows_v.at[b],
                    out_hbm.at[pl.ds(row0, _C), pl.ds(w * _EMB, _EMB)],
                    sem_w[b],
                )

    def wb_drain(b):
        pltpu.make_async_copy(
            rows_v.at[b],
            out_hbm.at[pl.ds(0, _C), pl.ds(0, _EMB)], sem_w[b],
        ).wait()

    for j in range(_K + 1):
        prefetch_idx(j, j)
    for j in range(_K):
        fire_gather(j, j)

    def group(gg, carry):
        for phase in range(_NBUF):
            i = gg * _NBUF + phase
            bi = phase
            bj = (phase + _K) % _NBUF
            bp = (phase + _K + 1) % _NBUF
            j = i + _K

            @pl.when(i + _K + 1 < _G)
            def _():
                prefetch_idx(i + _K + 1, bp)

            @pl.when(j < _G)
            def _():
                @pl.when(j >= _NBUF)
                def _():
                    wb_drain(bj)

                fire_gather(j, bj)

            pltpu.make_async_copy(
                table_hbm.at[idx_v.at[pl.ds(bi * _C, _C)]], rows_v.at[bi], sem_g[bi]
            ).wait()
            wb_start(i, bi)
        return carry

    lax.fori_loop(0, _G // _NBUF, group, 0)

    for b in range(_NBUF):
        wb_drain(b)


def _transpose_body(x_ref, o_ref):
    xt = jnp.transpose(x_ref[0], (1, 0))
    o_ref[...] = xt.reshape(4, _EMB, _BATCH)


def _transpose_slabs(y):
    return pl.pallas_call(
        _transpose_body,
        grid=(_TG,),
        in_specs=[pl.BlockSpec((1, _BATCH, 4 * _EMB), lambda g: (g, 0, 0))],
        out_specs=pl.BlockSpec((4, _EMB, _BATCH), lambda g: (g, 0, 0)),
        out_shape=jax.ShapeDtypeStruct((_HIST, _EMB, _BATCH), jnp.float32),
    )(y)


def kernel(input, weight):
    idx = jnp.transpose(input).reshape(_B).astype(jnp.int32)
    z = _gather_interleaved(idx, weight)
    out3 = _transpose_slabs(z.reshape(_TG, _BATCH, 4 * _EMB))
    return out3.transpose(2, 0, 1)
